# baseline (speedup 1.0000x reference)
import functools

import jax
import jax.numpy as jnp
from jax import lax
from jax.experimental import pallas as pl
from jax.experimental.pallas import tpu as pltpu
from jax.experimental.pallas import tpu_sc as plsc

BATCH = 16384
DIM = 32
LANES = 128
PACK = LANES // DIM
IDX_ROWS = 128
IDX_COLS = 128
ROWS_PER_TILE = 4
NUM_WORKERS = 32


CHUNK = 64
NCHUNKS = (ROWS_PER_TILE * IDX_COLS) // CHUNK


def _sc_gather_build():
  mesh = plsc.VectorSubcoreMesh(core_axis_name="c", subcore_axis_name="s")
  out_sh = jax.ShapeDtypeStruct((IDX_ROWS, IDX_COLS, LANES), jnp.float32)
  gbuf = pltpu.VMEM((CHUNK, LANES), jnp.float32)

  @functools.partial(
      pl.kernel,
      mesh=mesh,
      out_type=[out_sh, out_sh, out_sh, out_sh],
      scratch_types=[
          pltpu.VMEM((ROWS_PER_TILE, IDX_COLS), jnp.int32),
          pltpu.VMEM((ROWS_PER_TILE, IDX_COLS), jnp.int32),
          gbuf, gbuf, gbuf, gbuf,
          gbuf, gbuf, gbuf, gbuf,
          pltpu.SemaphoreType.DMA,
          pltpu.SemaphoreType.DMA,
      ],
  )
  def sc_gather(user_hbm, item_hbm, mfu_hbm, mfi_hbm, mlu_hbm, mli_hbm,
                o_mfu, o_mfi, o_mlu, o_mli,
                idx_u, idx_i,
                a0, a1, a2, a3, b0, b1, b2, b3, sem_g, sem_w):
    wid = lax.axis_index("s") * 2 + lax.axis_index("c")
    base = wid * ROWS_PER_TILE
    pltpu.sync_copy(user_hbm.at[pl.ds(base, ROWS_PER_TILE)], idx_u)
    pltpu.sync_copy(item_hbm.at[pl.ds(base, ROWS_PER_TILE)], idx_i)

    tabs = (mfu_hbm, mfi_hbm, mlu_hbm, mli_hbm)
    outs = (o_mfu, o_mfi, o_mlu, o_mli)
    idxs = (idx_u, idx_i, idx_u, idx_i)
    bufs = ((a0, a1, a2, a3), (b0, b1, b2, b3))

    def idx_slice(t, j):
      return idxs[t].at[j // 2, pl.ds((j % 2) * CHUNK, CHUNK)]

    def out_slice(t, j):
      return outs[t].at[base + j // 2, pl.ds((j % 2) * CHUNK, CHUNK)]

    def fire(j, bset):
      return [
          pltpu.async_copy(tabs[t].at[idx_slice(t, j)], bset[t], sem_g)
          for t in range(4)
      ]

    def drain(j, bset, gcopies):
      wcopies = []
      for t in range(4):
        gcopies[t].wait()
        wcopies.append(pltpu.async_copy(bset[t], out_slice(t, j), sem_w))
      return wcopies

    g = fire(0, bufs[0])
    pending_w = []
    for j in range(NCHUNKS):
      for w in pending_w:
        w.wait()
      nxt = fire(j + 1, bufs[(j + 1) % 2]) if j + 1 < NCHUNKS else None
      pending_w = drain(j, bufs[j % 2], g)
      g = nxt
    for w in pending_w:
      w.wait()

  return sc_gather


_SC_GATHER_CACHE = []


def _sc_gather(*args):
  if not _SC_GATHER_CACHE:
    _SC_GATHER_CACHE.append(_sc_gather_build())
  return _SC_GATHER_CACHE[0](*args)


TC_BLK = 2048


def _select(lines, sel):
  out = jnp.zeros((lines.shape[0], DIM), jnp.float32)
  for o in range(PACK):
    out = out + jnp.where(sel == o, lines[:, o * DIM:(o + 1) * DIM], 0.0)
  return out


def _tc_body(su, si, mfu, mfi, mlu, mli, w1, b1r, w2, b2r, wp, bpr, out):
  f32 = jnp.float32
  sel_u = su[...]
  sel_i = si[...]
  u = _select(mlu[...], sel_u)
  i = _select(mli[...], sel_i)
  w1m = w1[...]
  dn = (((1,), (1,)), ((), ()))
  x = (lax.dot_general(u, w1m[:, :DIM], dn, preferred_element_type=f32)
       + lax.dot_general(i, w1m[:, DIM:], dn, preferred_element_type=f32)
       + b1r[...])
  h = jnp.maximum(x, 0.0)
  h2 = jnp.maximum(
      lax.dot_general(h, w2[...], dn, preferred_element_type=f32) + b2r[...],
      0.0)
  mfp = _select(mfu[...], sel_u) * _select(mfi[...], sel_i)
  wpv = wp[...]
  s = (jnp.sum(mfp * wpv[:, :DIM], axis=1)
       + jnp.sum(h2 * wpv[:, DIM:], axis=1) + bpr[0])
  out[...] = s


def _tc_mlp(sel_u, sel_i, mf_u, mf_i, mlp_u, mlp_i, W1, b1, W2, b2, Wp, bp):
  grid = (BATCH // TC_BLK,)
  sel_spec = pl.BlockSpec((TC_BLK, 1), lambda g: (g, 0))
  row_spec = pl.BlockSpec((TC_BLK, LANES), lambda g: (g, 0))
  full = lambda shape: pl.BlockSpec(shape, lambda g: tuple(0 for _ in shape))
  return pl.pallas_call(
      _tc_body,
      grid=grid,
      in_specs=[
          sel_spec, sel_spec,
          row_spec, row_spec, row_spec, row_spec,
          full((64, 64)),
          full((1, 64)),
          full((32, 64)),
          full((1, 32)),
          full((1, 64)),
          pl.BlockSpec(memory_space=pltpu.SMEM),
      ],
      out_specs=pl.BlockSpec((TC_BLK,), lambda g: (g,)),
      out_shape=jax.ShapeDtypeStruct((BATCH,), jnp.float32),
  )(sel_u, sel_i, mf_u, mf_i, mlp_u, mlp_i, W1, b1.reshape(1, 64), W2,
    b2.reshape(1, 32), Wp, bp)


def kernel(user, item, mf_user_emb, mf_item_emb, mlp_user_emb, mlp_item_emb,
           W1, b1, W2, b2, Wp, bp):
  user32 = user.astype(jnp.int32)
  item32 = item.astype(jnp.int32)
  uhi = (user32 // PACK).reshape(IDX_ROWS, IDX_COLS)
  ihi = (item32 // PACK).reshape(IDX_ROWS, IDX_COLS)
  n4 = mf_user_emb.shape[0] // PACK
  r4 = lambda t: t.reshape(n4, LANES)
  mf_u, mf_i, mlp_u, mlp_i = _sc_gather(
      uhi, ihi, r4(mf_user_emb), r4(mf_item_emb), r4(mlp_user_emb),
      r4(mlp_item_emb))
  rl = lambda a: a.reshape(BATCH, LANES)
  sel_u = (user32 % PACK).reshape(BATCH, 1)
  sel_i = (item32 % PACK).reshape(BATCH, 1)
  return _tc_mlp(sel_u, sel_i, rl(mf_u), rl(mf_i), rl(mlp_u), rl(mlp_i),
                 W1, b1, W2, b2, Wp, bp)


# speedup vs baseline: 3.0697x; 3.0697x over previous
import functools

import jax
import jax.numpy as jnp
from jax import lax
from jax.experimental import pallas as pl
from jax.experimental.pallas import tpu as pltpu
from jax.experimental.pallas import tpu_sc as plsc

BATCH = 16384
DIM = 32
N_ROWS = 1000000
NUM_WORKERS = 32
B_PER_W = BATCH // NUM_WORKERS
EPW = DIM * B_PER_W
LANES = 16

BLK_L = 2048
FULL_BLOCKS = (N_ROWS // BLK_L // 2 // 2) * 2
TAIL_START = 2 * FULL_BLOCKS * BLK_L
TAIL_L = N_ROWS - TAIL_START


DT_K = 100
DT_L = DT_K * 128
DT_NC = -(-N_ROWS // DT_L)
OUT_ROWS = DT_NC * 4 * DT_K * 8
FLAT_N = OUT_ROWS * 128


def _tc_detile(tabT):
  grid = (DT_NC,)

  def body(t_ref, o_ref):
    x = t_ref[...].reshape(4, 8, DT_K, 128)
    o_ref[...] = x.transpose(2, 0, 1, 3).reshape(4 * DT_K * 8, 128)

  return pl.pallas_call(
      body,
      grid=grid,
      in_specs=[pl.BlockSpec((DIM, DT_L), lambda c: (0, c))],
      out_specs=pl.BlockSpec((4 * DT_K * 8, 128), lambda c: (c, 0)),
      out_shape=jax.ShapeDtypeStruct((OUT_ROWS, 128), jnp.float32),
  )(tabT).reshape(FLAT_N)


def _sc_gather_build():
  mesh = plsc.VectorSubcoreMesh(core_axis_name="c", subcore_axis_name="s")
  out_sh = jax.ShapeDtypeStruct((NUM_WORKERS, EPW), jnp.float32)
  ebuf = pltpu.VMEM((EPW,), jnp.int32)
  dbuf = pltpu.VMEM((EPW,), jnp.float32)

  @functools.partial(
      pl.kernel,
      mesh=mesh,
      out_type=[out_sh, out_sh, out_sh, out_sh],
      scratch_types=[
          pltpu.VMEM((B_PER_W,), jnp.int32),
          pltpu.VMEM((B_PER_W,), jnp.int32),
          ebuf, ebuf,
          dbuf, dbuf, dbuf, dbuf,
          pltpu.SemaphoreType.DMA,
      ],
  )
  def sc_gather(user_hbm, item_hbm, mfu_hbm, mfi_hbm, mlu_hbm, mli_hbm,
                o_mfu, o_mfi, o_mlu, o_mli,
                idx_u, idx_i, eidx_u, eidx_i, d0, d1, d2, d3, sem):
    wid = lax.axis_index("s") * 2 + lax.axis_index("c")
    base = wid * B_PER_W
    pltpu.sync_copy(user_hbm.at[pl.ds(base, B_PER_W)], idx_u)
    pltpu.sync_copy(item_hbm.at[pl.ds(base, B_PER_W)], idx_i)

    for j in range(B_PER_W // LANES):
      src = pl.ds(j * LANES, LANES)
      iu = idx_u[src]
      ii = idx_i[src]
      bu = ((iu >> 7) << 12) + (iu & 127)
      bi = ((ii >> 7) << 12) + (ii & 127)
      for d in range(DIM):
        off = (d // 8) * 1024 + (d % 8) * 128
        dst = pl.ds(d * B_PER_W + j * LANES, LANES)
        eidx_u[dst] = bu + off
        eidx_i[dst] = bi + off

    copies = [
        pltpu.async_copy(mfu_hbm.at[eidx_u], d0, sem),
        pltpu.async_copy(mfi_hbm.at[eidx_i], d1, sem),
        pltpu.async_copy(mlu_hbm.at[eidx_u], d2, sem),
        pltpu.async_copy(mli_hbm.at[eidx_i], d3, sem),
    ]
    for c in copies:
      c.wait()
    for buf, out in ((d0, o_mfu), (d1, o_mfi), (d2, o_mlu), (d3, o_mli)):
      pltpu.sync_copy(buf, out.at[wid])

  return sc_gather


_BUILD_CACHE = {}


def _get(name, builder):
  if name not in _BUILD_CACHE:
    _BUILD_CACHE[name] = builder()
  return _BUILD_CACHE[name]


def _tc_body(mfu, mfi, mlu, mli, w1, b1c, w2, b2c, wpa, wpb, bpr, out):
  f32 = jnp.float32
  u = mlu[0]
  i = mli[0]
  w1m = w1[...]
  dn = (((1,), (0,)), ((), ()))
  x = (lax.dot_general(w1m[:, :DIM], u, dn, preferred_element_type=f32)
       + lax.dot_general(w1m[:, DIM:], i, dn, preferred_element_type=f32)
       + b1c[...])
  h = jnp.maximum(x, 0.0)
  h2 = jnp.maximum(
      lax.dot_general(w2[...], h, dn, preferred_element_type=f32) + b2c[...],
      0.0)
  mfp = mfu[0] * mfi[0]
  s = (jnp.sum(mfp * wpa[...], axis=0) + jnp.sum(h2 * wpb[...], axis=0)
       + bpr[0])
  out[...] = s


def _tc_mlp(mf_u, mf_i, mlp_u, mlp_i, W1, b1, W2, b2, Wp, bp):
  grid = (NUM_WORKERS,)
  blk_spec = pl.BlockSpec((1, DIM, B_PER_W), lambda g: (g, 0, 0))
  full = lambda shape: pl.BlockSpec(shape, lambda g: tuple(0 for _ in shape))
  return pl.pallas_call(
      _tc_body,
      grid=grid,
      in_specs=[
          blk_spec, blk_spec, blk_spec, blk_spec,
          full((64, 64)),
          full((64, 1)),
          full((32, 64)),
          full((32, 1)),
          full((32, 1)),
          full((32, 1)),
          pl.BlockSpec(memory_space=pltpu.SMEM),
      ],
      out_specs=pl.BlockSpec((B_PER_W,), lambda g: (g,)),
      out_shape=jax.ShapeDtypeStruct((BATCH,), jnp.float32),
  )(mf_u, mf_i, mlp_u, mlp_i, W1, b1.reshape(64, 1), W2, b2.reshape(32, 1),
    Wp[0, :DIM].reshape(DIM, 1), Wp[0, DIM:].reshape(DIM, 1), bp)


def kernel(user, item, mf_user_emb, mf_item_emb, mlp_user_emb, mlp_item_emb,
           W1, b1, W2, b2, Wp, bp):
  user32 = user.astype(jnp.int32)
  item32 = item.astype(jnp.int32)
  f0 = _tc_detile(mf_user_emb.T)
  f1 = _tc_detile(mf_item_emb.T)
  f2 = _tc_detile(mlp_user_emb.T)
  f3 = _tc_detile(mlp_item_emb.T)
  gather = _get("gather", _sc_gather_build)
  mf_u, mf_i, mlp_u, mlp_i = gather(user32, item32, f0, f1, f2, f3)
  r = lambda a: a.reshape(NUM_WORKERS, DIM, B_PER_W)
  return _tc_mlp(r(mf_u), r(mf_i), r(mlp_u), r(mlp_i), W1, b1, W2, b2, Wp,
                 bp)
